# baseline (speedup 1.0000x reference)
import functools

import jax
import jax.numpy as jnp
from jax import lax
from jax.experimental import pallas as pl
from jax.experimental.pallas import tpu as pltpu
from jax.experimental.pallas import tpu_sc as plsc


def _build(B, V):
    info = plsc.get_sparse_core_info()
    NC, NS = info.num_cores, info.num_subcores
    NW = NC * NS
    b_per_w = B // NW
    CHUNK = 16
    n_chunks = b_per_w // CHUNK

    mesh = plsc.VectorSubcoreMesh(core_axis_name="c", subcore_axis_name="s")

    @functools.partial(
        pl.kernel,
        mesh=mesh,
        out_type=jax.ShapeDtypeStruct((2, 8, B), jnp.float32),
        scratch_types=[
            pltpu.VMEM((b_per_w,), jnp.int32),
            pltpu.VMEM((2, 8, CHUNK * 128), jnp.float32),
            pltpu.VMEM((2, 8, b_per_w), jnp.float32),
            pltpu.SemaphoreType.DMA,
        ],
        compiler_params=pltpu.CompilerParams(needs_layout_passes=False),
    )
    def gather_kernel(idx_hbm, tab_hbm, out_hbm, idx_v, buf_v, rows_v, sem):
        wid = lax.axis_index("s") * NC + lax.axis_index("c")
        base = wid * b_per_w
        lane = lax.iota(jnp.int32, 16)
        pltpu.sync_copy(idx_hbm.at[wid], idx_v)

        def chunk_body(chunk, carry):
            c0 = chunk * CHUNK

            vec = idx_v[pl.ds(c0, 16)]
            for j in range(16):
                q = pl.multiple_of(
                    (vec[j] >> jnp.int32(7)) * jnp.int32(128), 128
                )
                s = (vec[j] >> jnp.int32(4)) & jnp.int32(7)
                for sb in range(8):
                    @pl.when(s == sb)
                    def _():
                        pltpu.async_copy(
                            tab_hbm.at[:, :, pl.ds(q, 128)]
                                   .at[:, :, pl.ds(sb * 16, 16)],
                            buf_v.at[:, :, pl.ds(j * 128, 128)]
                                 .at[:, :, pl.ds(sb * 16, 16)],
                            sem,
                        )
            for j in range(16):
                pltpu.make_async_copy(
                    tab_hbm.at[:, :, pl.ds(0, 128)]
                           .at[:, :, pl.ds(0, 16)],
                    buf_v.at[:, :, pl.ds(j * 128, 128)]
                         .at[:, :, pl.ds(0, 16)],
                    sem,
                ).wait()

            low = vec & jnp.int32(127)
            pos = lane * 128 + low
            for ti in range(2):
                for cc in range(8):
                    vals = plsc.load_gather(
                        buf_v, [lane * 0 + ti, lane * 0 + cc, pos]
                    )
                    rows_v[ti, cc, pl.ds(c0, 16)] = vals
            return carry

        lax.fori_loop(0, n_chunks, chunk_body, 0)

        for ti in range(2):
            pltpu.sync_copy(
                rows_v.at[ti], out_hbm.at[ti, :, pl.ds(base, b_per_w)]
            )

    return gather_kernel


def kernel(g, h, table):
    B = h.shape[0]
    V = table.shape[0]
    info = plsc.get_sparse_core_info()
    NW = info.num_cores * info.num_subcores
    idx = h.astype(jnp.int32).reshape(NW, B // NW)
    tab3 = table.T.reshape(2, 8, V)
    out3 = _build(B, V)(idx, tab3)
    return out3.reshape(16, B).T


# speedup vs baseline: 1.4663x; 1.4663x over previous
import functools

import jax
import jax.numpy as jnp
from jax import lax
from jax.experimental import pallas as pl
from jax.experimental.pallas import tpu as pltpu
from jax.experimental.pallas import tpu_sc as plsc


def _build(B, V):
    info = plsc.get_sparse_core_info()
    NC, NS = info.num_cores, info.num_subcores
    NW = NC * NS
    b_per_w = B // NW
    CHUNK = 16
    n_chunks = b_per_w // CHUNK

    mesh = plsc.VectorSubcoreMesh(core_axis_name="c", subcore_axis_name="s")

    @functools.partial(
        pl.kernel,
        mesh=mesh,
        out_type=jax.ShapeDtypeStruct((2, 8, B), jnp.float32),
        scratch_types=[
            pltpu.VMEM((b_per_w,), jnp.int32),
            pltpu.VMEM((16, CHUNK * 128), jnp.float32),
            pltpu.VMEM((16, CHUNK * 128), jnp.float32),
            pltpu.VMEM((2, 8, b_per_w), jnp.float32),
            pltpu.SemaphoreType.DMA,
            pltpu.SemaphoreType.DMA,
        ],
        compiler_params=pltpu.CompilerParams(needs_layout_passes=False),
    )
    def gather_kernel(idx_hbm, tab_hbm, out_hbm,
                      idx_v, buf_a, buf_b, rows_v, sem_a, sem_b):
        wid = lax.axis_index("s") * NC + lax.axis_index("c")
        base = wid * b_per_w
        lane = lax.iota(jnp.int32, 16)
        pltpu.sync_copy(idx_hbm.at[wid], idx_v)

        def fire(g, buf, sem):
            vec = idx_v[pl.ds(g * CHUNK, 16)]
            cps = []
            for j in range(16):
                q = pl.multiple_of(
                    (vec[j] >> jnp.int32(7)) * jnp.int32(128), 128
                )
                cps.append(pltpu.async_copy(
                    tab_hbm.at[:, pl.ds(q, 128)].at[lane],
                    buf.at[:, pl.ds(j * 128, 128)],
                    sem,
                ))
            return cps

        def drain(cps):
            for cp in cps:
                cp.wait()

        def extract(g, buf):
            vec = idx_v[pl.ds(g * CHUNK, 16)]
            pos = lane * 128 + (vec & jnp.int32(127))
            for ti in range(2):
                for cc in range(8):
                    vals = plsc.load_gather(
                        buf, [lane * 0 + (ti * 8 + cc), pos]
                    )
                    rows_v[ti, cc, pl.ds(g * CHUNK, 16)] = vals

        def body(gp, carry):
            g0 = gp * 2
            cps_a = fire(g0, buf_a, sem_a)
            cps_b = fire(g0 + 1, buf_b, sem_b)
            drain(cps_a)
            extract(g0, buf_a)
            drain(cps_b)
            extract(g0 + 1, buf_b)
            return carry

        lax.fori_loop(0, n_chunks // 2, body, 0)

        for ti in range(2):
            pltpu.sync_copy(
                rows_v.at[ti], out_hbm.at[ti, :, pl.ds(base, b_per_w)]
            )

    return gather_kernel


def kernel(g, h, table):
    B = h.shape[0]
    V = table.shape[0]
    info = plsc.get_sparse_core_info()
    NW = info.num_cores * info.num_subcores
    idx = h.astype(jnp.int32).reshape(NW, B // NW)
    tab16 = table.T
    out3 = _build(B, V)(idx, tab16)
    return out3.reshape(16, B).T


# speedup vs baseline: 1.6043x; 1.0941x over previous
import functools

import jax
import jax.numpy as jnp
from jax import lax
from jax.experimental import pallas as pl
from jax.experimental.pallas import tpu as pltpu
from jax.experimental.pallas import tpu_sc as plsc


def _build(B, V):
    info = plsc.get_sparse_core_info()
    NC, NS = info.num_cores, info.num_subcores
    NW = NC * NS
    b_per_w = B // NW
    CHUNK = 16
    n_chunks = b_per_w // CHUNK

    mesh = plsc.VectorSubcoreMesh(core_axis_name="c", subcore_axis_name="s")

    @functools.partial(
        pl.kernel,
        mesh=mesh,
        out_type=jax.ShapeDtypeStruct((2, 8, B), jnp.float32),
        scratch_types=[
            pltpu.VMEM((b_per_w,), jnp.int32),
            pltpu.VMEM((16,), jnp.int32),
            pltpu.VMEM((16, CHUNK * 128), jnp.float32),
            pltpu.VMEM((16, CHUNK * 128), jnp.float32),
            pltpu.VMEM((2, 8, CHUNK * 128), jnp.float32),
            pltpu.VMEM((2, 8, b_per_w), jnp.float32),
            pltpu.SemaphoreType.DMA,
            pltpu.SemaphoreType.DMA,
            pltpu.SemaphoreType.DMA,
        ],
        compiler_params=pltpu.CompilerParams(needs_layout_passes=False),
    )
    def gather_kernel(idx_hbm, tab3_hbm, out_hbm,
                      idx_v, plane8_v, buf_a, buf_b, buf_p, rows_v,
                      sem_a, sem_b, sem_p):
        wid = lax.axis_index("s") * NC + lax.axis_index("c")
        base = wid * b_per_w
        lane = lax.iota(jnp.int32, 16)
        plane8_v[pl.ds(0, 16)] = lane & jnp.int32(7)
        pltpu.sync_copy(idx_hbm.at[wid], idx_v)

        def fire(g, buf, sem):
            vec = idx_v[pl.ds(g * CHUNK, 16)]
            cps = []
            for j in range(16):
                q = pl.multiple_of(
                    (vec[j] >> jnp.int32(7)) * jnp.int32(128), 128
                )
                for ti in range(2):
                    cps.append(pltpu.async_copy(
                        tab3_hbm.at[ti]
                                .at[:, pl.ds(q, 128)]
                                .at[plane8_v.at[pl.ds(0, 8)]],
                        buf.at[pl.ds(ti * 8, 8), pl.ds(j * 128, 128)],
                        sem,
                    ))
            return cps

        def drain(cps):
            for cp in cps:
                cp.wait()

        def extract(g, buf):
            vec = idx_v[pl.ds(g * CHUNK, 16)]
            pos = lane * 128 + (vec & jnp.int32(127))
            for ti in range(2):
                for cc in range(8):
                    vals = plsc.load_gather(
                        buf, [lane * 0 + (ti * 8 + cc), pos]
                    )
                    rows_v[ti, cc, pl.ds(g * CHUNK, 16)] = vals

        def fire_plain(g, buf, sem):
            vec = idx_v[pl.ds(g * CHUNK, 16)]
            for j in range(16):
                q = pl.multiple_of(
                    (vec[j] >> jnp.int32(7)) * jnp.int32(128), 128
                )
                s = (vec[j] >> jnp.int32(4)) & jnp.int32(7)
                for sb in range(8):
                    @pl.when(s == sb)
                    def _():
                        pltpu.async_copy(
                            tab3_hbm.at[:, :, pl.ds(q, 128)]
                                    .at[:, :, pl.ds(sb * 16, 16)],
                            buf.at[:, :, pl.ds(j * 128, 128)]
                               .at[:, :, pl.ds(sb * 16, 16)],
                            sem,
                        )

        def drain_plain(buf, sem):
            for j in range(16):
                pltpu.make_async_copy(
                    tab3_hbm.at[:, :, pl.ds(0, 128)]
                            .at[:, :, pl.ds(0, 16)],
                    buf.at[:, :, pl.ds(j * 128, 128)]
                       .at[:, :, pl.ds(0, 16)],
                    sem,
                ).wait()

        def extract_p(g, buf):
            vec = idx_v[pl.ds(g * CHUNK, 16)]
            pos = lane * 128 + (vec & jnp.int32(127))
            for ti in range(2):
                for cc in range(8):
                    vals = plsc.load_gather(
                        buf, [lane * 0 + ti, lane * 0 + cc, pos]
                    )
                    rows_v[ti, cc, pl.ds(g * CHUNK, 16)] = vals

        n_plain = 10
        n_stream_pairs = (n_chunks - n_plain) // 2

        def body(it, carry):
            g0 = it * 2
            cps_a = fire(g0, buf_a, sem_a)
            cps_b = fire(g0 + 1, buf_b, sem_b)
            pc = 2 * n_stream_pairs + it

            @pl.when(it < n_plain)
            def _():
                fire_plain(pc, buf_p, sem_p)

            @pl.when(it < n_plain)
            def _():
                drain_plain(buf_p, sem_p)
                extract_p(pc, buf_p)

            drain(cps_a)
            extract(g0, buf_a)
            drain(cps_b)
            extract(g0 + 1, buf_b)
            return carry

        lax.fori_loop(0, n_stream_pairs, body, 0)

        for ti in range(2):
            pltpu.sync_copy(
                rows_v.at[ti], out_hbm.at[ti, :, pl.ds(base, b_per_w)]
            )

    return gather_kernel


def kernel(g, h, table):
    B = h.shape[0]
    V = table.shape[0]
    info = plsc.get_sparse_core_info()
    NW = info.num_cores * info.num_subcores
    idx = h.astype(jnp.int32).reshape(NW, B // NW)
    tab3 = table.T.reshape(2, 8, V)
    out3 = _build(B, V)(idx, tab3)
    return out3.reshape(16, B).T
